# Initial kernel scaffold; baseline (speedup 1.0000x reference)
import jax
import jax.numpy as jnp
from jax import lax
from jax.experimental import pallas as pl
from jax.experimental.pallas import tpu as pltpu, tpu_sc as plsc

B = 128
V = 65536
CODE_LEN = 16
NS = 16
GROUP = 16
LEVELS_PER_W = CODE_LEN // 2


def _softplus(u):
    a = jnp.abs(u)
    e = jnp.exp(-a)
    t = e / (2.0 + e)
    t2 = t * t
    poly = 1.0 + t2 * (1.0 / 3.0 + t2 * (1.0 / 5.0 + t2 * (
        1.0 / 7.0 + t2 * (1.0 / 9.0 + t2 * (1.0 / 11.0)))))
    return jnp.maximum(u, 0.0) + 2.0 * t * poly


def _body(scores_hbm, ci_hbm, out_hbm,
          ci_v, idx_v, vals_v, acc_v, all_v, out_v, shared, sem):
    sid = lax.axis_index("s")
    g = sid // 2
    l = sid % 2

    pltpu.sync_copy(ci_hbm.at[pl.ds(g * GROUP, GROUP)], ci_v)
    c = ci_v[...]

    lane = lax.iota(jnp.int32, GROUP)
    row_base = (g * GROUP + lane) * V

    for j in range(LEVELS_PER_W):
        jj = l * LEVELS_PER_W + j
        d = (c >> (CODE_LEN - jj)) + (1 << jj)
        idx_v[pl.ds(GROUP * j, GROUP)] = row_base + d - 1

    pltpu.async_copy(scores_hbm.at[idx_v], vals_v, sem).wait()

    acc = jnp.zeros((GROUP,), jnp.float32)
    for j in range(LEVELS_PER_W):
        jj = l * LEVELS_PER_W + j
        s = vals_v[pl.ds(GROUP * j, GROUP)]
        bit = (c >> (CODE_LEN - 1 - jj)) & 1
        u = jnp.where(bit == 1, s, -s)
        acc = acc + _softplus(u)
    acc_v[...] = acc

    pltpu.sync_copy(acc_v, shared.at[sid])
    plsc.subcore_barrier()

    @pl.when(sid == 0)
    def _():
        pltpu.sync_copy(shared, all_v)
        tot = all_v[0, :]
        for r in range(1, NS):
            tot = tot + all_v[r, :]
        loss = lax.reduce_sum_p.bind(tot, axes=(0,)) * (1.0 / B)
        out_v[...] = jnp.full((GROUP,), loss, jnp.float32)
        pltpu.sync_copy(out_v, out_hbm)


def kernel(scores, class_indices):
    mesh = plsc.VectorSubcoreMesh(
        core_axis_name="c", subcore_axis_name="s", num_cores=1)
    run = pl.kernel(
        _body,
        out_type=jax.ShapeDtypeStruct((GROUP,), jnp.float32),
        mesh=mesh,
        scratch_types=[
            pltpu.VMEM((GROUP,), jnp.int32),
            pltpu.VMEM((GROUP * LEVELS_PER_W,), jnp.int32),
            pltpu.VMEM((GROUP * LEVELS_PER_W,), jnp.float32),
            pltpu.VMEM((GROUP,), jnp.float32),
            pltpu.VMEM((NS, GROUP), jnp.float32),
            pltpu.VMEM((GROUP,), jnp.float32),
            pltpu.VMEM_SHARED((NS, GROUP), jnp.float32),
            pltpu.SemaphoreType.DMA,
        ],
    )
    out = run(scores.reshape(-1), class_indices)
    return out[0]


# speedup vs baseline: 2.7047x; 2.7047x over previous
import jax
import jax.numpy as jnp
from jax import lax
from jax.experimental import pallas as pl
from jax.experimental.pallas import tpu as pltpu, tpu_sc as plsc

B = 128
V = 65536
CODE_LEN = 16
NS = 16
GROUP = 16
LEVELS_PER_W = CODE_LEN // 2


def _softplus(u):
    a = jnp.abs(u)
    e = jnp.exp(-a)
    t = e / (2.0 + e)
    t2 = t * t
    poly = 1.0 + t2 * (1.0 / 3.0 + t2 * (1.0 / 5.0 + t2 * (
        1.0 / 7.0 + t2 * (1.0 / 9.0 + t2 * (1.0 / 11.0)))))
    return jnp.maximum(u, 0.0) + 2.0 * t * poly


def _body(scores_hbm, ci_hbm, out_hbm, ci_v, idx_v, vals_v, acc_v, sem):
    sid = lax.axis_index("s")
    g = sid // 2
    l = sid % 2

    pltpu.sync_copy(ci_hbm.at[pl.ds(g * GROUP, GROUP)], ci_v)
    c = ci_v[...]

    lane = lax.iota(jnp.int32, GROUP)
    row_base = (g * GROUP + lane) * V

    for j in range(LEVELS_PER_W):
        jj = l * LEVELS_PER_W + j
        d = (c >> (CODE_LEN - jj)) + (1 << jj)
        idx_v[pl.ds(GROUP * j, GROUP)] = row_base + d - 1

    pltpu.async_copy(scores_hbm.at[idx_v], vals_v, sem).wait()

    acc = jnp.zeros((GROUP,), jnp.float32)
    for j in range(LEVELS_PER_W):
        jj = l * LEVELS_PER_W + j
        s = vals_v[pl.ds(GROUP * j, GROUP)]
        bit = (c >> (CODE_LEN - 1 - jj)) & 1
        u = jnp.where(bit == 1, s, -s)
        acc = acc + _softplus(u)
    acc_v[...] = acc

    pltpu.sync_copy(acc_v, out_hbm.at[sid])


def _reduce_body(p_ref, o_ref):
    o_ref[...] = jnp.sum(p_ref[...], keepdims=True) * (1.0 / B)


def kernel(scores, class_indices):
    mesh = plsc.VectorSubcoreMesh(
        core_axis_name="c", subcore_axis_name="s", num_cores=1)
    run = pl.kernel(
        _body,
        out_type=jax.ShapeDtypeStruct((NS, GROUP), jnp.float32),
        mesh=mesh,
        scratch_types=[
            pltpu.VMEM((GROUP,), jnp.int32),
            pltpu.VMEM((GROUP * LEVELS_PER_W,), jnp.int32),
            pltpu.VMEM((GROUP * LEVELS_PER_W,), jnp.float32),
            pltpu.VMEM((GROUP,), jnp.float32),
            pltpu.SemaphoreType.DMA,
        ],
    )
    partials = run(scores.reshape(-1), class_indices)

    reduce = pl.pallas_call(
        _reduce_body,
        out_shape=jax.ShapeDtypeStruct((1, 1), jnp.float32),
    )
    return reduce(partials)[0, 0]


# speedup vs baseline: 6.2973x; 2.3283x over previous
import jax
import jax.numpy as jnp
from jax import lax
from jax.experimental import pallas as pl
from jax.experimental.pallas import tpu as pltpu, tpu_sc as plsc

B = 128
V = 65536
CODE_LEN = 16
NS = 16
GROUP = 16
LEVELS_PER_W = CODE_LEN // 2
NROW = B * V // 128


def _softplus(u):
    a = jnp.abs(u)
    e = jnp.exp(-a)
    t = e / (2.0 + e)
    t2 = t * t
    poly = 1.0 + t2 * (1.0 / 3.0 + t2 * (1.0 / 5.0 + t2 * (
        1.0 / 7.0 + t2 * (1.0 / 9.0 + t2 * (1.0 / 11.0)))))
    return jnp.maximum(u, 0.0) + 2.0 * t * poly


def _body(scores_hbm, ci_hbm, out_hbm, ci_v, idx_v, col_v, vals_v, acc_v, sem):
    sid = lax.axis_index("s")
    g = sid // 2
    l = sid % 2

    pltpu.sync_copy(ci_hbm.at[pl.ds(g * GROUP, GROUP)], ci_v)
    c = ci_v[...]

    lane = lax.iota(jnp.int32, GROUP)
    b = g * GROUP + lane
    row_b = ((b >> 3) << 12) + (b & 7)

    for j in range(LEVELS_PER_W):
        jj = l * LEVELS_PER_W + j
        q = (c >> (CODE_LEN - jj)) + (1 << jj) - 1
        idx_v[pl.ds(GROUP * j, GROUP)] = row_b + ((q >> 7) << 3)
        col_v[pl.ds(GROUP * j, GROUP)] = q & 127

    rows = scores_hbm.reshape(NROW, 128)
    pltpu.async_copy(rows.at[idx_v], vals_v, sem).wait()

    acc = jnp.zeros((GROUP,), jnp.float32)
    for j in range(LEVELS_PER_W):
        jj = l * LEVELS_PER_W + j
        s = plsc.load_gather(
            vals_v, [GROUP * j + lane, col_v[pl.ds(GROUP * j, GROUP)]])
        bit = (c >> (CODE_LEN - 1 - jj)) & 1
        u = jnp.where(bit == 1, s, -s)
        acc = acc + _softplus(u)
    acc_v[...] = acc

    pltpu.sync_copy(acc_v, out_hbm.at[sid])


def _reduce_body(p_ref, o_ref):
    o_ref[...] = jnp.sum(p_ref[...], keepdims=True) * (1.0 / B)


def kernel(scores, class_indices):
    mesh = plsc.VectorSubcoreMesh(
        core_axis_name="c", subcore_axis_name="s", num_cores=1)
    run = pl.kernel(
        _body,
        out_type=jax.ShapeDtypeStruct((NS, GROUP), jnp.float32),
        mesh=mesh,
        compiler_params=pltpu.CompilerParams(needs_layout_passes=False),
        scratch_types=[
            pltpu.VMEM((GROUP,), jnp.int32),
            pltpu.VMEM((GROUP * LEVELS_PER_W,), jnp.int32),
            pltpu.VMEM((GROUP * LEVELS_PER_W,), jnp.int32),
            pltpu.VMEM((GROUP * LEVELS_PER_W, 128), jnp.float32),
            pltpu.VMEM((GROUP,), jnp.float32),
            pltpu.SemaphoreType.DMA,
        ],
    )
    tiled = scores.reshape(16, 8, 512, 128).transpose(0, 2, 1, 3)
    partials = run(tiled, class_indices)

    reduce = pl.pallas_call(
        _reduce_body,
        out_shape=jax.ShapeDtypeStruct((1, 1), jnp.float32),
    )
    return reduce(partials)[0, 0]
